# Initial kernel scaffold; baseline (speedup 1.0000x reference)
import jax
import jax.numpy as jnp
from jax.experimental import pallas as pl


def kernel(x, w1, b1, w2, b2):
    raise NotImplementedError("write your pallas kernel here")


# speedup vs baseline: 1.3896x; 1.3896x over previous
import functools

import jax
import jax.numpy as jnp
from jax.experimental import pallas as pl
from jax.experimental.pallas import tpu as pltpu


def _ca_fused_kernel(x_ref, w1_ref, b1_ref, w2_ref, b2_ref, o_ref, *, inv_hw):
    x = x_ref[...]
    s = jnp.sum(x, axis=1, keepdims=True, dtype=jnp.float32) * inv_hw
    h = jnp.dot(w1_ref[...], s,
                preferred_element_type=jnp.float32) + b1_ref[...]
    h = jnp.maximum(h, 0.0)
    z = jnp.dot(w2_ref[...], h,
                preferred_element_type=jnp.float32) + b2_ref[...]
    y = jax.nn.sigmoid(z)
    o_ref[...] = (x * y.astype(x.dtype)).astype(o_ref.dtype)


def kernel(x, w1, b1, w2, b2):
    N, C, H, W = x.shape
    C8 = w1.shape[0]
    HW = H * W

    hw_pad = pl.cdiv(HW, 128) * 128
    x_flat = x.reshape(N, C, HW)
    if hw_pad != HW:
        x_flat = jnp.pad(x_flat, ((0, 0), (0, 0), (0, hw_pad - HW)))

    out = pl.pallas_call(
        functools.partial(_ca_fused_kernel, inv_hw=1.0 / float(HW)),
        out_shape=jax.ShapeDtypeStruct((N, C, hw_pad), x.dtype),
        grid_spec=pltpu.PrefetchScalarGridSpec(
            num_scalar_prefetch=0,
            grid=(N,),
            in_specs=[
                pl.BlockSpec((pl.Squeezed(), C, hw_pad), lambda n: (n, 0, 0)),
                pl.BlockSpec((C8, C), lambda n: (0, 0)),
                pl.BlockSpec((C8, 1), lambda n: (0, 0)),
                pl.BlockSpec((C, C8), lambda n: (0, 0)),
                pl.BlockSpec((C, 1), lambda n: (0, 0)),
            ],
            out_specs=pl.BlockSpec((pl.Squeezed(), C, hw_pad),
                                   lambda n: (n, 0, 0)),
        ),
        compiler_params=pltpu.CompilerParams(
            dimension_semantics=("parallel",),
            vmem_limit_bytes=64 << 20),
    )(x_flat, w1, b1.reshape(C8, 1), w2, b2.reshape(C, 1))

    if hw_pad != HW:
        out = out[:, :, :HW]
    return out.reshape(N, C, H, W)


# speedup vs baseline: 5.2160x; 3.7535x over previous
import functools

import jax
import jax.numpy as jnp
from jax.experimental import pallas as pl
from jax.experimental.pallas import tpu as pltpu


def _ca_fused_kernel(x_ref, w1_ref, b1_ref, w2_ref, b2_ref, o_ref, *, inv_hw):
    x = x_ref[...]
    C = x.shape[0]
    s = jnp.sum(x, axis=(1, 2), keepdims=True,
                dtype=jnp.float32).reshape(C, 1) * inv_hw
    h = jnp.dot(w1_ref[...], s,
                preferred_element_type=jnp.float32) + b1_ref[...]
    h = jnp.maximum(h, 0.0)
    z = jnp.dot(w2_ref[...], h,
                preferred_element_type=jnp.float32) + b2_ref[...]
    y = jax.nn.sigmoid(z).reshape(C, 1, 1)
    o_ref[...] = (x * y.astype(x.dtype)).astype(o_ref.dtype)


def kernel(x, w1, b1, w2, b2):
    N, C, H, W = x.shape
    C8 = w1.shape[0]

    return pl.pallas_call(
        functools.partial(_ca_fused_kernel, inv_hw=1.0 / float(H * W)),
        out_shape=jax.ShapeDtypeStruct((N, C, H, W), x.dtype),
        grid_spec=pltpu.PrefetchScalarGridSpec(
            num_scalar_prefetch=0,
            grid=(N,),
            in_specs=[
                pl.BlockSpec((pl.Squeezed(), C, H, W),
                             lambda n: (n, 0, 0, 0)),
                pl.BlockSpec((C8, C), lambda n: (0, 0)),
                pl.BlockSpec((C8, 1), lambda n: (0, 0)),
                pl.BlockSpec((C, C8), lambda n: (0, 0)),
                pl.BlockSpec((C, 1), lambda n: (0, 0)),
            ],
            out_specs=pl.BlockSpec((pl.Squeezed(), C, H, W),
                                   lambda n: (n, 0, 0, 0)),
        ),
        compiler_params=pltpu.CompilerParams(
            dimension_semantics=("parallel",),
            vmem_limit_bytes=64 << 20),
    )(x, w1, b1.reshape(C8, 1), w2, b2.reshape(C, 1))


# speedup vs baseline: 5.2411x; 1.0048x over previous
import functools

import jax
import jax.numpy as jnp
from jax.experimental import pallas as pl
from jax.experimental.pallas import tpu as pltpu


def _ca_fused_kernel(x_ref, w1_ref, b1_ref, w2_ref, b2_ref, o_ref, *, inv_hw):
    x = x_ref[...]
    C = x.shape[0]
    s = jnp.sum(x, axis=(1, 2), keepdims=True,
                dtype=jnp.float32).reshape(C, 1) * inv_hw
    h = jnp.dot(w1_ref[...], s,
                preferred_element_type=jnp.float32) + b1_ref[...]
    h = jnp.maximum(h, 0.0)
    z = jnp.dot(w2_ref[...], h,
                preferred_element_type=jnp.float32) + b2_ref[...]
    y = jax.nn.sigmoid(z).reshape(C, 1, 1)
    o_ref[...] = (x * y.astype(x.dtype)).astype(o_ref.dtype)


def kernel(x, w1, b1, w2, b2):
    N, C, H, W = x.shape
    C8 = w1.shape[0]

    grid = (N,)
    semantics = ("parallel",)
    x_map = lambda n: (n, 0, 0, 0)
    w_map = lambda n: (0, 0)

    return pl.pallas_call(
        functools.partial(_ca_fused_kernel, inv_hw=1.0 / float(H * W)),
        out_shape=jax.ShapeDtypeStruct((N, C, H, W), x.dtype),
        grid_spec=pltpu.PrefetchScalarGridSpec(
            num_scalar_prefetch=0,
            grid=grid,
            in_specs=[
                pl.BlockSpec((pl.Squeezed(), C, H, W), x_map),
                pl.BlockSpec((C8, C), w_map),
                pl.BlockSpec((C8, 1), w_map),
                pl.BlockSpec((C, C8), w_map),
                pl.BlockSpec((C, 1), w_map),
            ],
            out_specs=pl.BlockSpec((pl.Squeezed(), C, H, W), x_map),
        ),
        compiler_params=pltpu.CompilerParams(
            dimension_semantics=semantics,
            vmem_limit_bytes=64 << 20),
    )(x, w1, b1.reshape(C8, 1), w2, b2.reshape(C, 1))


# speedup vs baseline: 5.7390x; 1.0950x over previous
import functools

import jax
import jax.numpy as jnp
from jax.experimental import pallas as pl
from jax.experimental.pallas import tpu as pltpu


def _ca_fused_kernel(x_ref, w1t_ref, b1_ref, w2t_ref, b2_ref, o_ref, *, inv_hw):
    x = x_ref[...]
    s = jnp.sum(x, axis=(2, 3), dtype=jnp.float32) * inv_hw
    h = jnp.dot(s, w1t_ref[...],
                preferred_element_type=jnp.float32) + b1_ref[...]
    h = jnp.maximum(h, 0.0)
    z = jnp.dot(h, w2t_ref[...],
                preferred_element_type=jnp.float32) + b2_ref[...]
    y = jax.nn.sigmoid(z)
    o_ref[...] = (x * y[:, :, None, None].astype(x.dtype)).astype(o_ref.dtype)


def kernel(x, w1, b1, w2, b2):
    N, C, H, W = x.shape
    C8 = w1.shape[0]

    NB = 2 if N % 2 == 0 else 1

    return pl.pallas_call(
        functools.partial(_ca_fused_kernel, inv_hw=1.0 / float(H * W)),
        out_shape=jax.ShapeDtypeStruct((N, C, H, W), x.dtype),
        grid_spec=pltpu.PrefetchScalarGridSpec(
            num_scalar_prefetch=0,
            grid=(N // NB,),
            in_specs=[
                pl.BlockSpec((NB, C, H, W), lambda n: (n, 0, 0, 0)),
                pl.BlockSpec((C, C8), lambda n: (0, 0)),
                pl.BlockSpec((1, C8), lambda n: (0, 0)),
                pl.BlockSpec((C8, C), lambda n: (0, 0)),
                pl.BlockSpec((1, C), lambda n: (0, 0)),
            ],
            out_specs=pl.BlockSpec((NB, C, H, W), lambda n: (n, 0, 0, 0)),
        ),
        compiler_params=pltpu.CompilerParams(
            dimension_semantics=("parallel",),
            vmem_limit_bytes=100 << 20),
    )(x, w1.T, b1.reshape(1, C8), w2.T, b2.reshape(1, C))
